# baseline (speedup 1.0000x reference)
import functools
import jax
import jax.numpy as jnp
from jax import lax
from jax.experimental import pallas as pl
from jax.experimental.pallas import tpu as pltpu
from jax.experimental.pallas import tpu_sc as plsc

B, NF, Fm, H, NC = 8, 4096, 8, 128, 512
FH = Fm * H
NK = 4
CH = NF // NK
NWORK = 32
SEG_PER = NC // NWORK
CKN = 16


def _scores_body(x_ref, w_ref, out_ref):
    xb = x_ref[0]
    xm = jnp.sum(xb, axis=1) * (1.0 / Fm)
    s = lax.dot_general(xm, w_ref[...], (((1,), (0,)), ((), ())),
                        preferred_element_type=jnp.float32)
    out_ref[0, 0, :] = s[:, 0]


def _weights_body(s_ref, seg_ref, w_out_ref, st_out_ref):
    s = s_ref[:, 0, :]
    m = jnp.max(s, axis=1, keepdims=True)
    e = jnp.exp(s - m)
    seg = seg_ref[...]
    iota_c = lax.broadcasted_iota(jnp.int32, (NC, 1), 0)
    M = (seg == iota_c).astype(jnp.float32)
    denom = lax.dot_general(e, M, (((1,), (1,)), ((), ())),
                            preferred_element_type=jnp.float32)
    recip = jnp.where(denom > 0, 1.0 / denom, 0.0)
    gath = lax.dot_general(recip, M, (((1,), (0,)), ((), ())),
                           preferred_element_type=jnp.float32)
    w_out_ref[...] = e * gath

    ones_row = jnp.ones((1, NF), jnp.float32)
    hist = lax.dot_general(ones_row, M, (((1,), (1,)), ((), ())),
                           preferred_element_type=jnp.float32)
    r = lax.broadcasted_iota(jnp.int32, (NC, NC), 0)
    c = lax.broadcasted_iota(jnp.int32, (NC, NC), 1)
    T2 = (r < c).astype(jnp.float32)
    starts = lax.dot_general(hist, T2, (((1,), (0,)), ((), ())),
                             preferred_element_type=jnp.float32)
    st_out_ref[...] = starts.astype(jnp.int32)


def _sc_pool_body(x_hbm, w_hbm, st_hbm, out_hbm, st_v, wv, xbuf, accv, zbuf):
    cid = lax.axis_index("c")
    sid = lax.axis_index("s")
    wid = sid * 2 + cid
    c0 = wid * SEG_PER
    pltpu.sync_copy(st_hbm.at[pl.ds(c0, 32)], st_v)
    pltpu.sync_copy(w_hbm, wv)
    for f in range(Fm):
        for k in range(H // 16):
            zbuf[f, pl.ds(k * 16, 16)] = jnp.zeros((16,), jnp.float32)

    def batch_body(b, _):
        def seg_body(cl, _):
            iota16 = lax.broadcasted_iota(jnp.int32, (16,), 0)
            stv = plsc.load_gather(st_v, [cl + iota16])
            lo = stv[0]
            hi = stv[1]
            cnt = hi - lo
            nch = (cnt + CKN - 1) // CKN

            def chunk_body(j, _):
                m_int = lo + j * CKN
                m = jnp.minimum(m_int, NF - CKN)
                pltpu.sync_copy(x_hbm.at[b, pl.ds(m, CKN)], xbuf)
                n_vec = m + lax.broadcasted_iota(jnp.int32, (CKN,), 0)
                valid = jnp.logical_and(n_vec >= m_int, n_vec < hi)
                wraw = plsc.load_gather(wv, [jnp.full((CKN,), b, jnp.int32), n_vec])
                wvec = jnp.where(valid, wraw, 0.0)
                wns = [wvec[i] for i in range(CKN)]
                first = j == 0
                for f in range(Fm):
                    accs = []
                    for t in range(H // 16):
                        av = accv[f, pl.ds(t * 16, 16)]
                        accs.append(jnp.where(first, 0.0, av))
                    for i in range(CKN):
                        for t in range(H // 16):
                            accs[t] = accs[t] + wns[i] * xbuf[i, f, pl.ds(t * 16, 16)]
                    for t in range(H // 16):
                        accv[f, pl.ds(t * 16, 16)] = accs[t]
                return 0

            lax.fori_loop(0, nch, chunk_body, 0)
            cg = c0 + cl

            @pl.when(cnt == 0)
            def _():
                pltpu.sync_copy(zbuf, out_hbm.at[b, cg])

            @pl.when(cnt > 0)
            def _():
                pltpu.sync_copy(accv, out_hbm.at[b, cg])

            return 0

        lax.fori_loop(0, SEG_PER, seg_body, 0)
        return 0

    lax.fori_loop(0, B, batch_body, 0)


_sc_pool = functools.partial(
    pl.kernel,
    out_type=jax.ShapeDtypeStruct((B, NC, Fm, H), jnp.float32),
    mesh=plsc.VectorSubcoreMesh(core_axis_name="c", subcore_axis_name="s"),
    scratch_types=[
        pltpu.VMEM((32,), jnp.int32),
        pltpu.VMEM((B, NF), jnp.float32),
        pltpu.VMEM((CKN, Fm, H), jnp.float32),
        pltpu.VMEM((Fm, H), jnp.float32),
        pltpu.VMEM((Fm, H), jnp.float32),
    ],
    compiler_params=pltpu.CompilerParams(needs_layout_passes=False),
)(_sc_pool_body)


def kernel(x, segment_ids, W, b):
    del b
    seg2 = segment_ids.reshape(1, NF).astype(jnp.int32)

    scores = pl.pallas_call(
        _scores_body,
        grid=(B, NK),
        in_specs=[
            pl.BlockSpec((1, CH, Fm, H), lambda bi, ki: (bi, ki, 0, 0)),
            pl.BlockSpec((H, 1), lambda bi, ki: (0, 0)),
        ],
        out_specs=pl.BlockSpec((1, 1, CH), lambda bi, ki: (bi, 0, ki)),
        out_shape=jax.ShapeDtypeStruct((B, 1, NF), jnp.float32),
    )(x, W)

    wts, starts = pl.pallas_call(
        _weights_body,
        in_specs=[
            pl.BlockSpec((B, 1, NF), lambda: (0, 0, 0)),
            pl.BlockSpec((1, NF), lambda: (0, 0)),
        ],
        out_specs=[
            pl.BlockSpec((B, NF), lambda: (0, 0)),
            pl.BlockSpec((1, NC), lambda: (0, 0)),
        ],
        out_shape=[
            jax.ShapeDtypeStruct((B, NF), jnp.float32),
            jax.ShapeDtypeStruct((1, NC), jnp.int32),
        ],
    )(scores, seg2)

    starts_ext = jnp.concatenate(
        [starts[0], jnp.full((32,), NF, jnp.int32)])
    return _sc_pool(x, wts, starts_ext)


# speedup vs baseline: 1.3906x; 1.3906x over previous
import functools
import jax
import jax.numpy as jnp
from jax import lax
from jax.experimental import pallas as pl
from jax.experimental.pallas import tpu as pltpu
from jax.experimental.pallas import tpu_sc as plsc

B, NF, Fm, H, NC = 8, 4096, 8, 128, 512
FH = Fm * H
NK = 4
CH = NF // NK
NWORK = 32
SEG_PER = NC // NWORK
CKN = 16


def _scores_body(x_ref, w_ref, out_ref):
    xb = x_ref[0]
    xm = jnp.sum(xb, axis=1) * (1.0 / Fm)
    s = lax.dot_general(xm, w_ref[...], (((1,), (0,)), ((), ())),
                        preferred_element_type=jnp.float32)
    out_ref[0, 0, :] = s[:, 0]


def _weights_body(s_ref, seg_ref, w_out_ref, st_out_ref):
    s = s_ref[:, 0, :]
    m = jnp.max(s, axis=1, keepdims=True)
    e = jnp.exp(s - m)
    seg = seg_ref[...]
    iota_c = lax.broadcasted_iota(jnp.int32, (NC, 1), 0)
    M = (seg == iota_c).astype(jnp.float32)
    denom = lax.dot_general(e, M, (((1,), (1,)), ((), ())),
                            preferred_element_type=jnp.float32)
    recip = jnp.where(denom > 0, 1.0 / denom, 0.0)
    gath = lax.dot_general(recip, M, (((1,), (0,)), ((), ())),
                           preferred_element_type=jnp.float32)
    w_out_ref[...] = e * gath

    ones_row = jnp.ones((1, NF), jnp.float32)
    hist = lax.dot_general(ones_row, M, (((1,), (1,)), ((), ())),
                           preferred_element_type=jnp.float32)
    r = lax.broadcasted_iota(jnp.int32, (NC, NC), 0)
    c = lax.broadcasted_iota(jnp.int32, (NC, NC), 1)
    T2 = (r < c).astype(jnp.float32)
    starts = lax.dot_general(hist, T2, (((1,), (0,)), ((), ())),
                             preferred_element_type=jnp.float32)
    st_out_ref[...] = starts.astype(jnp.int32)


def _sc_pool_body(x_hbm, w_hbm, st_hbm, out_hbm,
                  st_v, wv, xbuf, pref, accb, xsem, osem):
    cid = lax.axis_index("c")
    sid = lax.axis_index("s")
    wid = sid * 2 + cid
    c0 = wid * SEG_PER
    pltpu.sync_copy(st_hbm.at[pl.ds(c0, 32)], st_v)
    stv_a = st_v[pl.ds(0, 16)]
    stv_b = st_v[pl.ds(16, 16)]
    n0 = stv_a[0]
    n1 = stv_b[0]
    a0 = (n0 // CKN) * CKN
    nch = jnp.maximum(1, (n1 - a0 + CKN - 1) // CKN)
    iota16 = lax.broadcasted_iota(jnp.int32, (16,), 0)

    def chunk_addr(j):
        return jnp.minimum(a0 + j * CKN, NF - CKN)

    def issue_x(b, j, par):
        pltpu.async_copy(x_hbm.at[b, pl.ds(chunk_addr(j), CKN)],
                         xbuf.at[par], xsem.at[par])

    def wait_x(b, par):
        pltpu.make_async_copy(x_hbm.at[b, pl.ds(0, CKN)],
                              xbuf.at[par], xsem.at[par]).wait()

    def wait_out():
        pltpu.make_async_copy(accb.at[0], out_hbm.at[0, 0], osem).wait()

    def batch_body(b, fl_in):
        pltpu.sync_copy(w_hbm.at[pl.ds(b * NF, NF)], wv)
        issue_x(b, 0, 0)

        def chunk_body(ck, carry):
            cur, slo, fl, fresh = carry
            par = lax.rem(ck, 2)
            wait_x(b, par)

            @pl.when(ck + 1 < nch)
            def _():
                issue_x(b, ck + 1, 1 - par)

            m = chunk_addr(ck)
            n_vec = m + iota16
            valid = jnp.logical_and(n_vec >= n0, n_vec < n1)
            wraw = plsc.load_gather(wv, [n_vec])
            wvec = jnp.where(valid, wraw, 0.0)
            wns = [wvec[i] for i in range(CKN)]
            for f in range(Fm):
                accs = [jnp.zeros((16,), jnp.float32) for _ in range(H // 16)]
                for t in range(H // 16):
                    pref[0, f, pl.ds(t * 16, 16)] = accs[t]
                for i in range(CKN):
                    xrow = xbuf.at[par]
                    for t in range(H // 16):
                        accs[t] = accs[t] + wns[i] * xrow[i, f, pl.ds(t * 16, 16)]
                    for t in range(H // 16):
                        pref[i + 1, f, pl.ds(t * 16, 16)] = accs[t]
            chunk_end = m + CKN

            def run_cond(st):
                return st[4] != 0

            def run_body(st):
                cur, slo, fl, fresh, _more = st
                shi = plsc.load_gather(st_v, [jnp.full((16,), cur + 1, jnp.int32)])[0]
                startl = jnp.clip(slo - m, 0, CKN)
                endl = jnp.clip(shi - m, 0, CKN)
                par_o = lax.rem(fl, 2)
                for f in range(Fm):
                    for t in range(H // 16):
                        pe = pref[endl, f, pl.ds(t * 16, 16)]
                        ps = pref[startl, f, pl.ds(t * 16, 16)]
                        av = accb[par_o, f, pl.ds(t * 16, 16)]
                        base = jnp.where(fresh != 0, 0.0, av)
                        accb[par_o, f, pl.ds(t * 16, 16)] = base + (pe - ps)
                complete = shi <= chunk_end

                @pl.when(complete)
                def _():
                    @pl.when(fl >= 1)
                    def _():
                        wait_out()
                    pltpu.async_copy(accb.at[par_o], out_hbm.at[b, c0 + cur], osem)

                cur2 = jnp.where(complete, cur + 1, cur)
                slo2 = jnp.where(complete, shi, slo)
                fl2 = jnp.where(complete, fl + 1, fl)
                fresh2 = jnp.where(complete, 1, 0)
                more = jnp.where(jnp.logical_and(complete, cur2 < SEG_PER), 1, 0)
                return (cur2, slo2, fl2, fresh2, more)

            cur, slo, fl, fresh, _ = lax.while_loop(
                run_cond, run_body, (cur, slo, fl, fresh, jnp.int32(1)))
            return (cur, slo, fl, fresh)

        _, _, fl_out, _ = lax.fori_loop(
            0, nch, chunk_body, (jnp.int32(0), n0, fl_in, jnp.int32(1)))
        return fl_out

    lax.fori_loop(0, B, batch_body, jnp.int32(0))
    wait_out()


_sc_pool = functools.partial(
    pl.kernel,
    out_type=jax.ShapeDtypeStruct((B, NC, Fm, H), jnp.float32),
    mesh=plsc.VectorSubcoreMesh(core_axis_name="c", subcore_axis_name="s"),
    scratch_types=[
        pltpu.VMEM((32,), jnp.int32),
        pltpu.VMEM((NF,), jnp.float32),
        pltpu.VMEM((2, CKN, Fm, H), jnp.float32),
        pltpu.VMEM((CKN + 1, Fm, H), jnp.float32),
        pltpu.VMEM((2, Fm, H), jnp.float32),
        pltpu.SemaphoreType.DMA((2,)),
        pltpu.SemaphoreType.DMA,
    ],
    compiler_params=pltpu.CompilerParams(needs_layout_passes=False),
)(_sc_pool_body)


def kernel(x, segment_ids, W, b):
    del b
    seg2 = segment_ids.reshape(1, NF).astype(jnp.int32)

    scores = pl.pallas_call(
        _scores_body,
        grid=(B, NK),
        in_specs=[
            pl.BlockSpec((1, CH, Fm, H), lambda bi, ki: (bi, ki, 0, 0)),
            pl.BlockSpec((H, 1), lambda bi, ki: (0, 0)),
        ],
        out_specs=pl.BlockSpec((1, 1, CH), lambda bi, ki: (bi, 0, ki)),
        out_shape=jax.ShapeDtypeStruct((B, 1, NF), jnp.float32),
    )(x, W)

    wts, starts = pl.pallas_call(
        _weights_body,
        in_specs=[
            pl.BlockSpec((B, 1, NF), lambda: (0, 0, 0)),
            pl.BlockSpec((1, NF), lambda: (0, 0)),
        ],
        out_specs=[
            pl.BlockSpec((B, NF), lambda: (0, 0)),
            pl.BlockSpec((1, NC), lambda: (0, 0)),
        ],
        out_shape=[
            jax.ShapeDtypeStruct((B, NF), jnp.float32),
            jax.ShapeDtypeStruct((1, NC), jnp.int32),
        ],
    )(scores, seg2)

    starts_ext = jnp.concatenate(
        [starts[0], jnp.full((32,), NF, jnp.int32)])
    return _sc_pool(x, wts.reshape(B * NF), starts_ext)


# speedup vs baseline: 2.6593x; 1.9123x over previous
import functools
import jax
import jax.numpy as jnp
from jax import lax
from jax.experimental import pallas as pl
from jax.experimental.pallas import tpu as pltpu
from jax.experimental.pallas import tpu_sc as plsc

B, NF, Fm, H, NC = 8, 4096, 8, 128, 512
FH = Fm * H
NK = 4
CH = NF // NK
NWORK = 32
SEG_PER = NC // NWORK
CKN = 16
KB = 5


def _scores_body(x_ref, w_ref, out_ref):
    xb = x_ref[0]
    s = lax.dot_general(xb[:, 0, :], w_ref[...], (((1,), (0,)), ((), ())),
                        preferred_element_type=jnp.float32)
    for f in range(1, Fm):
        s = s + lax.dot_general(xb[:, f, :], w_ref[...],
                                (((1,), (0,)), ((), ())),
                                preferred_element_type=jnp.float32)
    out_ref[0, 0, :] = s[:, 0] * (1.0 / Fm)


def _weights_body(s_ref, seg_ref, w_out_ref, st_out_ref):
    s = s_ref[:, 0, :]
    m = jnp.max(s, axis=1, keepdims=True)
    e = jnp.exp(s - m)
    seg = seg_ref[...]
    iota_c = lax.broadcasted_iota(jnp.int32, (NC, 1), 0)
    M = (seg == iota_c).astype(jnp.float32)
    denom = lax.dot_general(e, M, (((1,), (1,)), ((), ())),
                            preferred_element_type=jnp.float32)
    recip = jnp.where(denom > 0, 1.0 / denom, 0.0)
    gath = lax.dot_general(recip, M, (((1,), (0,)), ((), ())),
                           preferred_element_type=jnp.float32)
    w_out_ref[...] = e * gath

    ones_row = jnp.ones((1, NF), jnp.float32)
    hist = lax.dot_general(ones_row, M, (((1,), (1,)), ((), ())),
                           preferred_element_type=jnp.float32)
    r = lax.broadcasted_iota(jnp.int32, (NC, NC), 0)
    c = lax.broadcasted_iota(jnp.int32, (NC, NC), 1)
    T2 = (r < c).astype(jnp.float32)
    starts = lax.dot_general(hist, T2, (((1,), (0,)), ((), ())),
                             preferred_element_type=jnp.float32)
    st_out_ref[...] = starts.astype(jnp.int32)


def _pool_body(x_ref, w_ref, seg_ref, out_ref):
    k = pl.program_id(1)

    @pl.when(k == 0)
    def _():
        out_ref[...] = jnp.zeros_like(out_ref)

    seg = seg_ref[0]
    wts = w_ref[0]
    iota_c = lax.broadcasted_iota(jnp.int32, (NC, 1), 0)
    Mw = jnp.where(seg == iota_c, wts, 0.0)
    xc = x_ref[0]
    for f in range(Fm):
        out_ref[0, :, f, :] += lax.dot_general(
            Mw, xc[:, f, :], (((1,), (0,)), ((), ())),
            preferred_element_type=jnp.float32)


def _sc_pool_body(x_hbm, w_hbm, st_hbm, out_hbm,
                  st_v, wv, xbuf, pref, accb, xsem, osem):
    cid = lax.axis_index("c")
    sid = lax.axis_index("s")
    wid = sid * 2 + cid
    c0 = wid * SEG_PER
    pltpu.sync_copy(st_hbm.at[pl.ds(c0, 32)], st_v)
    stv_a = st_v[pl.ds(0, 16)]
    stv_b = st_v[pl.ds(16, 16)]
    n0 = stv_a[0]
    n1 = stv_b[0]
    a0 = (n0 // CKN) * CKN
    nch = jnp.maximum(1, (n1 - a0 + CKN - 1) // CKN)
    iota16 = lax.broadcasted_iota(jnp.int32, (16,), 0)

    def chunk_addr(j):
        return jnp.minimum(a0 + j * CKN, NF - CKN)

    def issue_x(b, j, par):
        pltpu.async_copy(x_hbm.at[b, pl.ds(chunk_addr(j), CKN)],
                         xbuf.at[par], xsem.at[par])

    def wait_x(b, par):
        pltpu.make_async_copy(x_hbm.at[b, pl.ds(0, CKN)],
                              xbuf.at[par], xsem.at[par]).wait()

    def wait_out():
        pltpu.make_async_copy(accb.at[0], out_hbm.at[0, 0], osem).wait()

    def batch_body(b, fl_in):
        pltpu.sync_copy(w_hbm.at[pl.ds(b * NF, NF)], wv)
        issue_x(b, 0, 0)

        def chunk_body(ck, carry):
            cur, slo, fl, fresh = carry
            par = lax.rem(ck, 2)
            wait_x(b, par)

            @pl.when(ck + 1 < nch)
            def _():
                issue_x(b, ck + 1, 1 - par)

            m = chunk_addr(ck)
            n_vec = m + iota16
            valid = jnp.logical_and(n_vec >= n0, n_vec < n1)
            wraw = plsc.load_gather(wv, [n_vec])
            wvec = jnp.where(valid, wraw, 0.0)
            wns = [wvec[i] for i in range(CKN)]
            for f in range(Fm):
                accs = [jnp.zeros((16,), jnp.float32) for _ in range(H // 16)]
                for t in range(H // 16):
                    pref[0, f, pl.ds(t * 16, 16)] = accs[t]
                for i in range(CKN):
                    xrow = xbuf.at[par]
                    for t in range(H // 16):
                        accs[t] = accs[t] + wns[i] * xrow[i, f, pl.ds(t * 16, 16)]
                    for t in range(H // 16):
                        pref[i + 1, f, pl.ds(t * 16, 16)] = accs[t]
            chunk_end = m + CKN

            def run_cond(st):
                return st[4] != 0

            def run_body(st):
                cur, slo, fl, fresh, _more = st
                shi = plsc.load_gather(st_v, [jnp.full((16,), cur + 1, jnp.int32)])[0]
                startl = jnp.clip(slo - m, 0, CKN)
                endl = jnp.clip(shi - m, 0, CKN)
                par_o = lax.rem(fl, 2)
                for f in range(Fm):
                    for t in range(H // 16):
                        pe = pref[endl, f, pl.ds(t * 16, 16)]
                        ps = pref[startl, f, pl.ds(t * 16, 16)]
                        av = accb[par_o, f, pl.ds(t * 16, 16)]
                        base = jnp.where(fresh != 0, 0.0, av)
                        accb[par_o, f, pl.ds(t * 16, 16)] = base + (pe - ps)
                complete = shi <= chunk_end

                @pl.when(complete)
                def _():
                    @pl.when(fl >= 1)
                    def _():
                        wait_out()
                    pltpu.async_copy(accb.at[par_o], out_hbm.at[b - KB, c0 + cur],
                                     osem)

                cur2 = jnp.where(complete, cur + 1, cur)
                slo2 = jnp.where(complete, shi, slo)
                fl2 = jnp.where(complete, fl + 1, fl)
                fresh2 = jnp.where(complete, 1, 0)
                more = jnp.where(jnp.logical_and(complete, cur2 < SEG_PER), 1, 0)
                return (cur2, slo2, fl2, fresh2, more)

            cur, slo, fl, fresh, _ = lax.while_loop(
                run_cond, run_body, (cur, slo, fl, fresh, jnp.int32(1)))
            return (cur, slo, fl, fresh)

        _, _, fl_out, _ = lax.fori_loop(
            0, nch, chunk_body, (jnp.int32(0), n0, fl_in, jnp.int32(1)))
        return fl_out

    lax.fori_loop(KB, B, batch_body, jnp.int32(0))
    wait_out()


_sc_pool = functools.partial(
    pl.kernel,
    out_type=jax.ShapeDtypeStruct((B - KB, NC, Fm, H), jnp.float32),
    mesh=plsc.VectorSubcoreMesh(core_axis_name="c", subcore_axis_name="s"),
    scratch_types=[
        pltpu.VMEM((32,), jnp.int32),
        pltpu.VMEM((NF,), jnp.float32),
        pltpu.VMEM((2, CKN, Fm, H), jnp.float32),
        pltpu.VMEM((CKN + 1, Fm, H), jnp.float32),
        pltpu.VMEM((2, Fm, H), jnp.float32),
        pltpu.SemaphoreType.DMA((2,)),
        pltpu.SemaphoreType.DMA,
    ],
    compiler_params=pltpu.CompilerParams(needs_layout_passes=False),
)(_sc_pool_body)


def kernel(x, segment_ids, W, b):
    del b
    seg2 = segment_ids.reshape(1, NF).astype(jnp.int32)

    scores = pl.pallas_call(
        _scores_body,
        grid=(B, NK),
        in_specs=[
            pl.BlockSpec((1, CH, Fm, H), lambda bi, ki: (bi, ki, 0, 0)),
            pl.BlockSpec((H, 1), lambda bi, ki: (0, 0)),
        ],
        out_specs=pl.BlockSpec((1, 1, CH), lambda bi, ki: (bi, 0, ki)),
        out_shape=jax.ShapeDtypeStruct((B, 1, NF), jnp.float32),
    )(x, W)

    wts, starts = pl.pallas_call(
        _weights_body,
        in_specs=[
            pl.BlockSpec((B, 1, NF), lambda: (0, 0, 0)),
            pl.BlockSpec((1, NF), lambda: (0, 0)),
        ],
        out_specs=[
            pl.BlockSpec((B, NF), lambda: (0, 0)),
            pl.BlockSpec((1, NC), lambda: (0, 0)),
        ],
        out_shape=[
            jax.ShapeDtypeStruct((B, NF), jnp.float32),
            jax.ShapeDtypeStruct((1, NC), jnp.int32),
        ],
    )(scores, seg2)

    starts_ext = jnp.concatenate(
        [starts[0], jnp.full((32,), NF, jnp.int32)])

    pooled_sc = _sc_pool(x, wts.reshape(B * NF), starts_ext)

    seg3 = segment_ids.reshape(NK, 1, CH).astype(jnp.int32)
    pooled_tc = pl.pallas_call(
        _pool_body,
        grid=(KB, NK),
        in_specs=[
            pl.BlockSpec((1, CH, Fm, H), lambda bi, ki: (bi, ki, 0, 0)),
            pl.BlockSpec((1, 1, CH), lambda bi, ki: (bi, 0, ki)),
            pl.BlockSpec((1, 1, CH), lambda bi, ki: (ki, 0, 0)),
        ],
        out_specs=pl.BlockSpec((1, NC, Fm, H), lambda bi, ki: (bi, 0, 0, 0)),
        out_shape=jax.ShapeDtypeStruct((KB, NC, Fm, H), jnp.float32),
    )(x, wts.reshape(B, 1, NF), seg3)

    return jnp.concatenate([pooled_tc, pooled_sc], axis=0)
